# baseline (speedup 1.0000x reference)
import functools

import jax
import jax.numpy as jnp
from jax import lax
from jax.experimental import pallas as pl
from jax.experimental.pallas import tpu as pltpu
from jax.experimental.pallas import tpu_sc as plsc

_NC = 2
_NS = 16
_NW = _NC * _NS
_L = 16

_POSE_W = 16


def _pose_table_body(rv_ref, tv_ref, out_ref):
    rx = rv_ref[0, :]
    ry = rv_ref[1, :]
    rz = rv_ref[2, :]
    theta = jnp.sqrt(rx * rx + ry * ry + rz * rz)
    safe = jnp.where(theta < 1e-8, jnp.float32(1.0), theta)
    inv = 1.0 / safe
    kx = rx * inv
    ky = ry * inv
    kz = rz * inv
    s = jnp.sin(safe)
    c = jnp.cos(safe)
    one_c = 1.0 - c
    kk = kx * kx + ky * ky + kz * kz
    r00 = 1.0 + one_c * (kx * kx - kk)
    r11 = 1.0 + one_c * (ky * ky - kk)
    r22 = 1.0 + one_c * (kz * kz - kk)
    xy = one_c * kx * ky
    xz = one_c * kx * kz
    yz = one_c * ky * kz
    skx = s * kx
    sky = s * ky
    skz = s * kz
    r01 = xy - skz
    r10 = xy + skz
    r02 = xz + sky
    r20 = xz - sky
    r12 = yz - skx
    r21 = yz + skx
    cols = [r00, r01, r02, r10, r11, r12, r20, r21, r22,
            tv_ref[0, :], tv_ref[1, :], tv_ref[2, :]]
    zero = jnp.zeros_like(rx)
    cols += [zero, zero, zero, zero]
    out_ref[...] = jnp.stack(cols, axis=0).T


def _build_pose_table(rv_t, tv_t, n_pad, block):
    grid = n_pad // block
    return pl.pallas_call(
        _pose_table_body,
        grid=(grid,),
        in_specs=[
            pl.BlockSpec((3, block), lambda i: (0, i)),
            pl.BlockSpec((3, block), lambda i: (0, i)),
        ],
        out_specs=pl.BlockSpec((block, _POSE_W), lambda i: (i, 0)),
        out_shape=jax.ShapeDtypeStruct((n_pad, _POSE_W), jnp.float32),
    )(rv_t, tv_t)


def _make_apply(m_points, chunk):
    assert m_points % chunk == 0 and chunk % 8 == 0
    n_chunks = m_points // chunk
    iters = (n_chunks + _NW - 1) // _NW
    n_pairs = (iters + 1) // 2
    n_grp = chunk // _L
    sub_offs = [(o, min(128, chunk - o)) for o in range(0, chunk, 128)]
    mesh = plsc.VectorSubcoreMesh(core_axis_name="c", subcore_axis_name="s")

    assert chunk % 128 == 0

    tiles = chunk // 128

    @functools.partial(
        pl.kernel,
        out_type=jax.ShapeDtypeStruct((m_points // 128, 4, 128), jnp.float32),
        mesh=mesh,
        compiler_params=pltpu.CompilerParams(
            needs_layout_passes=False, use_tc_tiling_on_sc=False),
        scratch_types=[
            pltpu.VMEM((2, chunk), jnp.int32),
            pltpu.VMEM((2, chunk, _POSE_W), jnp.float32),
            pltpu.VMEM((2, 3, chunk), jnp.float32),
            pltpu.VMEM((2, tiles, 4, 128), jnp.float32),
            pltpu.SemaphoreType.DMA,
            pltpu.SemaphoreType.DMA,
            pltpu.SemaphoreType.DMA,
            pltpu.SemaphoreType.DMA,
        ],
    )
    def apply(idx_hbm, x_hbm, y_hbm, z_hbm, poses_hbm, out_hbm,
              idx_v, poses_v, pts_v, out_v, gsem0, gsem1, osem0, osem1):
        wid = lax.axis_index("s") * _NC + lax.axis_index("c")
        lanes = lax.iota(jnp.int32, _L)
        col = [jnp.full((_L,), j, jnp.int32) for j in range(12)]
        gsem = [gsem0, gsem1]
        osem = [osem0, osem1]
        ins = [x_hbm, y_hbm, z_hbm]

        def fire_in(cid, b):
            base = cid * chunk
            pltpu.sync_copy(idx_hbm.at[pl.ds(base, chunk)], idx_v.at[b])
            for o, sz in sub_offs:
                pltpu.async_copy(
                    poses_hbm.at[idx_v.at[b].at[pl.ds(o, sz)]],
                    poses_v.at[b].at[pl.ds(o, sz)],
                    gsem[b],
                )
            for c in range(3):
                pltpu.async_copy(ins[c].at[pl.ds(base, chunk)],
                                 pts_v.at[b].at[c], gsem[b])

        def drain_in(b):
            for o, sz in sub_offs:
                pltpu.make_async_copy(
                    poses_hbm.at[pl.ds(0, sz)],
                    poses_v.at[b].at[pl.ds(o, sz)],
                    gsem[b],
                ).wait()
            for c in range(3):
                pltpu.make_async_copy(ins[c].at[pl.ds(0, chunk)],
                                      pts_v.at[b].at[c], gsem[b]).wait()

        def fire_out(cid, b):
            pltpu.async_copy(out_v.at[b],
                             out_hbm.at[pl.ds(cid * tiles, tiles)],
                             osem[b])

        def drain_out(b):
            pltpu.make_async_copy(out_v.at[b],
                                  out_hbm.at[pl.ds(0, tiles)],
                                  osem[b]).wait()

        def compute(b):
            pv = poses_v.at[b]
            xr = pts_v.at[b].at[0]
            yr = pts_v.at[b].at[1]
            zr = pts_v.at[b].at[2]
            ov = out_v.at[b]

            @plsc.parallel_loop(0, n_grp, unroll=4)
            def grp_body(g):
                rows = g * _L + lanes
                sp = pl.ds(g * _L, _L)
                t = g >> 3
                s = pl.ds((g & 7) * _L, _L)
                p = [plsc.load_gather(pv, [rows, col[j]]) for j in range(12)]
                x = xr[sp]
                y = yr[sp]
                z = zr[sp]
                ov[t, 0, s] = p[0] * x + p[1] * y + p[2] * z + p[9]
                ov[t, 1, s] = p[3] * x + p[4] * y + p[5] * z + p[10]
                ov[t, 2, s] = p[6] * x + p[7] * y + p[8] * z + p[11]

        def process(cid, b, prefetch_cid):
            @pl.when(cid < n_chunks)
            def _():
                @pl.when(prefetch_cid < n_chunks)
                def _():
                    fire_in(prefetch_cid, 1 - b)

                drain_in(b)

                @pl.when(cid >= 2 * _NW)
                def _():
                    drain_out(b)

                compute(b)
                fire_out(cid, b)

        fire_in(wid, 0)

        def pair_body(p, _):
            c_a = wid + (2 * p) * _NW
            process(c_a, 0, c_a + _NW)
            process(c_a + _NW, 1, c_a + 2 * _NW)
            return 0

        lax.fori_loop(0, n_pairs, pair_body, 0, unroll=False)
        drain_out(0)
        drain_out(1)

    return apply


def kernel(frame_indices, pts_3d, rvecs, tvecs):
    n = rvecs.shape[0]
    m = pts_3d.shape[0]
    block = 4096
    n_pad = ((n + block - 1) // block) * block
    rv_t = jnp.pad(rvecs, ((0, n_pad - n), (0, 0))).T
    tv_t = jnp.pad(tvecs, ((0, n_pad - n), (0, 0))).T
    poses = _build_pose_table(rv_t, tv_t, n_pad, block)

    chunk = 640
    apply = _make_apply(m, chunk)
    out4 = apply(frame_indices.astype(jnp.int32),
                 pts_3d[:, 0], pts_3d[:, 1], pts_3d[:, 2], poses)
    out = out4.transpose(0, 2, 1)
    return out.reshape(m, 4)[:, :3]


# speedup vs baseline: 1.1802x; 1.1802x over previous
import functools

import jax
import jax.numpy as jnp
from jax import lax
from jax.experimental import pallas as pl
from jax.experimental.pallas import tpu as pltpu
from jax.experimental.pallas import tpu_sc as plsc

_NC = 2
_NS = 16
_NW = _NC * _NS
_L = 16

_POSE_W = 16


def _pose_table_body(rv_ref, tv_ref, out_ref):
    rx = rv_ref[0, :]
    ry = rv_ref[1, :]
    rz = rv_ref[2, :]
    theta = jnp.sqrt(rx * rx + ry * ry + rz * rz)
    safe = jnp.where(theta < 1e-8, jnp.float32(1.0), theta)
    inv = 1.0 / safe
    kx = rx * inv
    ky = ry * inv
    kz = rz * inv
    s = jnp.sin(safe)
    c = jnp.cos(safe)
    one_c = 1.0 - c
    kk = kx * kx + ky * ky + kz * kz
    r00 = 1.0 + one_c * (kx * kx - kk)
    r11 = 1.0 + one_c * (ky * ky - kk)
    r22 = 1.0 + one_c * (kz * kz - kk)
    xy = one_c * kx * ky
    xz = one_c * kx * kz
    yz = one_c * ky * kz
    skx = s * kx
    sky = s * ky
    skz = s * kz
    r01 = xy - skz
    r10 = xy + skz
    r02 = xz + sky
    r20 = xz - sky
    r12 = yz - skx
    r21 = yz + skx
    cols = [r00, r01, r02, r10, r11, r12, r20, r21, r22,
            tv_ref[0, :], tv_ref[1, :], tv_ref[2, :]]
    zero = jnp.zeros_like(rx)
    cols += [zero, zero, zero, zero]
    out_ref[...] = jnp.stack(cols, axis=0).T


def _build_pose_table(rv_t, tv_t, n_pad, block):
    grid = n_pad // block
    return pl.pallas_call(
        _pose_table_body,
        grid=(grid,),
        in_specs=[
            pl.BlockSpec((3, block), lambda i: (0, i)),
            pl.BlockSpec((3, block), lambda i: (0, i)),
        ],
        out_specs=pl.BlockSpec((block, _POSE_W), lambda i: (i, 0)),
        out_shape=jax.ShapeDtypeStruct((n_pad, _POSE_W), jnp.float32),
    )(rv_t, tv_t)


def _make_apply(m_points, chunk):
    assert m_points % chunk == 0 and chunk % 8 == 0
    n_chunks = m_points // chunk
    iters = (n_chunks + _NW - 1) // _NW
    n_trip = (iters + 2) // 3
    n_grp = chunk // _L
    sub_offs = [(o, min(128, chunk - o)) for o in range(0, chunk, 128)]
    mesh = plsc.VectorSubcoreMesh(core_axis_name="c", subcore_axis_name="s")

    assert chunk % 128 == 0

    tiles = chunk // 128

    @functools.partial(
        pl.kernel,
        out_type=jax.ShapeDtypeStruct((m_points // 128, 4, 128), jnp.float32),
        mesh=mesh,
        compiler_params=pltpu.CompilerParams(
            needs_layout_passes=False, use_tc_tiling_on_sc=False),
        scratch_types=[
            pltpu.VMEM((3, chunk), jnp.int32),
            pltpu.VMEM((3, chunk, _POSE_W), jnp.float32),
            pltpu.VMEM((3, 3, chunk), jnp.float32),
            pltpu.VMEM((3, tiles, 4, 128), jnp.float32),
            [pltpu.SemaphoreType.DMA] * 3,
            [pltpu.SemaphoreType.DMA] * 3,
            [pltpu.SemaphoreType.DMA] * 3,
        ],
    )
    def apply(idx_hbm, x_hbm, y_hbm, z_hbm, poses_hbm, out_hbm,
              idx_v, poses_v, pts_v, out_v, isem, gsem, osem):
        wid = lax.axis_index("s") * _NC + lax.axis_index("c")
        lanes = lax.iota(jnp.int32, _L)
        col = [jnp.full((_L,), j, jnp.int32) for j in range(12)]
        ins = [x_hbm, y_hbm, z_hbm]

        def fire_idx(cid, b):
            pltpu.async_copy(idx_hbm.at[pl.ds(cid * chunk, chunk)],
                             idx_v.at[b], isem[b])

        def drain_idx(b):
            pltpu.make_async_copy(idx_hbm.at[pl.ds(0, chunk)], idx_v.at[b],
                                  isem[b]).wait()

        def fire_in(cid, b):
            base = cid * chunk
            for o, sz in sub_offs:
                pltpu.async_copy(
                    poses_hbm.at[idx_v.at[b].at[pl.ds(o, sz)]],
                    poses_v.at[b].at[pl.ds(o, sz)],
                    gsem[b],
                )
            for c in range(3):
                pltpu.async_copy(ins[c].at[pl.ds(base, chunk)],
                                 pts_v.at[b].at[c], gsem[b])

        def drain_in(b):
            for o, sz in sub_offs:
                pltpu.make_async_copy(
                    poses_hbm.at[pl.ds(0, sz)],
                    poses_v.at[b].at[pl.ds(o, sz)],
                    gsem[b],
                ).wait()
            for c in range(3):
                pltpu.make_async_copy(ins[c].at[pl.ds(0, chunk)],
                                      pts_v.at[b].at[c], gsem[b]).wait()

        def fire_out(cid, b):
            pltpu.async_copy(out_v.at[b],
                             out_hbm.at[pl.ds(cid * tiles, tiles)],
                             osem[b])

        def drain_out(b):
            pltpu.make_async_copy(out_v.at[b],
                                  out_hbm.at[pl.ds(0, tiles)],
                                  osem[b]).wait()

        def compute(b):
            pv = poses_v.at[b]
            xr = pts_v.at[b].at[0]
            yr = pts_v.at[b].at[1]
            zr = pts_v.at[b].at[2]
            ov = out_v.at[b]

            def grp_body(g, _):
                rows = g * _L + lanes
                sp = pl.ds(g * _L, _L)
                t = g >> 3
                s = pl.ds((g & 7) * _L, _L)
                p = [plsc.load_gather(pv, [rows, col[j]]) for j in range(12)]
                x = xr[sp]
                y = yr[sp]
                z = zr[sp]
                ov[t, 0, s] = p[0] * x + p[1] * y + p[2] * z + p[9]
                ov[t, 1, s] = p[3] * x + p[4] * y + p[5] * z + p[10]
                ov[t, 2, s] = p[6] * x + p[7] * y + p[8] * z + p[11]
                return 0

            lax.fori_loop(0, n_grp, grp_body, 0, unroll=False)

        def process(cid, b):
            @pl.when(cid < n_chunks)
            def _():
                @pl.when(cid + 2 * _NW < n_chunks)
                def _():
                    fire_idx(cid + 2 * _NW, (b + 2) % 3)

                @pl.when(cid + _NW < n_chunks)
                def _():
                    drain_idx((b + 1) % 3)
                    fire_in(cid + _NW, (b + 1) % 3)

                drain_in(b)

                @pl.when(cid >= 3 * _NW)
                def _():
                    drain_out(b)

                compute(b)
                fire_out(cid, b)

        fire_idx(wid, 0)
        drain_idx(0)
        fire_in(wid, 0)
        fire_idx(wid + _NW, 1)

        def trip_body(p, _):
            c_a = wid + (3 * p) * _NW
            process(c_a, 0)
            process(c_a + _NW, 1)
            process(c_a + 2 * _NW, 2)
            return 0

        lax.fori_loop(0, n_trip, trip_body, 0, unroll=False)
        drain_out(0)
        drain_out(1)
        drain_out(2)

    return apply


def kernel(frame_indices, pts_3d, rvecs, tvecs):
    n = rvecs.shape[0]
    m = pts_3d.shape[0]
    block = 4096
    n_pad = ((n + block - 1) // block) * block
    rv_t = jnp.pad(rvecs, ((0, n_pad - n), (0, 0))).T
    tv_t = jnp.pad(tvecs, ((0, n_pad - n), (0, 0))).T
    poses = _build_pose_table(rv_t, tv_t, n_pad, block)

    chunk = 640
    apply = _make_apply(m, chunk)
    out4 = apply(frame_indices.astype(jnp.int32),
                 pts_3d[:, 0], pts_3d[:, 1], pts_3d[:, 2], poses)
    out = out4.transpose(0, 2, 1)
    return out.reshape(m, 4)[:, :3]


# speedup vs baseline: 1.2350x; 1.0465x over previous
import functools

import jax
import jax.numpy as jnp
from jax import lax
from jax.experimental import pallas as pl
from jax.experimental.pallas import tpu as pltpu
from jax.experimental.pallas import tpu_sc as plsc

_NC = 2
_NS = 16
_NW = _NC * _NS
_L = 16

_POSE_W = 16


def _pose_table_body(rv_ref, tv_ref, out_ref):
    rx = rv_ref[0, :]
    ry = rv_ref[1, :]
    rz = rv_ref[2, :]
    theta = jnp.sqrt(rx * rx + ry * ry + rz * rz)
    safe = jnp.where(theta < 1e-8, jnp.float32(1.0), theta)
    inv = 1.0 / safe
    kx = rx * inv
    ky = ry * inv
    kz = rz * inv
    s = jnp.sin(safe)
    c = jnp.cos(safe)
    one_c = 1.0 - c
    kk = kx * kx + ky * ky + kz * kz
    r00 = 1.0 + one_c * (kx * kx - kk)
    r11 = 1.0 + one_c * (ky * ky - kk)
    r22 = 1.0 + one_c * (kz * kz - kk)
    xy = one_c * kx * ky
    xz = one_c * kx * kz
    yz = one_c * ky * kz
    skx = s * kx
    sky = s * ky
    skz = s * kz
    r01 = xy - skz
    r10 = xy + skz
    r02 = xz + sky
    r20 = xz - sky
    r12 = yz - skx
    r21 = yz + skx
    cols = [r00, r01, r02, r10, r11, r12, r20, r21, r22,
            tv_ref[0, :], tv_ref[1, :], tv_ref[2, :]]
    zero = jnp.zeros_like(rx)
    cols += [zero, zero, zero, zero]
    out_ref[...] = jnp.stack(cols, axis=0).T


def _build_pose_table(rv_t, tv_t, n_pad, block):
    grid = n_pad // block
    return pl.pallas_call(
        _pose_table_body,
        grid=(grid,),
        in_specs=[
            pl.BlockSpec((3, block), lambda i: (0, i)),
            pl.BlockSpec((3, block), lambda i: (0, i)),
        ],
        out_specs=pl.BlockSpec((block, _POSE_W), lambda i: (i, 0)),
        out_shape=jax.ShapeDtypeStruct((n_pad, _POSE_W), jnp.float32),
    )(rv_t, tv_t)


def _make_apply(m_points, chunk):
    assert m_points % chunk == 0 and chunk % 8 == 0
    n_chunks = m_points // chunk
    iters = (n_chunks + _NW - 1) // _NW
    n_quad = (iters + 3) // 4
    n_grp = chunk // _L
    sub_offs = [(o, min(128, chunk - o)) for o in range(0, chunk, 128)]
    mesh = plsc.VectorSubcoreMesh(core_axis_name="c", subcore_axis_name="s")

    assert chunk % 128 == 0

    tiles = chunk // 128

    @functools.partial(
        pl.kernel,
        out_type=jax.ShapeDtypeStruct((m_points // 128, 4, 128), jnp.float32),
        mesh=mesh,
        compiler_params=pltpu.CompilerParams(
            needs_layout_passes=False, use_tc_tiling_on_sc=False),
        scratch_types=[
            pltpu.VMEM((4, chunk), jnp.int32),
            pltpu.VMEM((4, chunk, _POSE_W), jnp.float32),
            pltpu.VMEM((4, 3, chunk), jnp.float32),
            pltpu.VMEM((4, tiles, 4, 128), jnp.float32),
            [pltpu.SemaphoreType.DMA] * 4,
            [pltpu.SemaphoreType.DMA] * 4,
            [pltpu.SemaphoreType.DMA] * 4,
        ],
    )
    def apply(idx_hbm, x_hbm, y_hbm, z_hbm, poses_hbm, out_hbm,
              idx_v, poses_v, pts_v, out_v, isem, gsem, osem):
        wid = lax.axis_index("s") * _NC + lax.axis_index("c")
        lanes = lax.iota(jnp.int32, _L)
        col = [jnp.full((_L,), j, jnp.int32) for j in range(12)]
        ins = [x_hbm, y_hbm, z_hbm]

        def fire_idx(cid, b):
            pltpu.async_copy(idx_hbm.at[pl.ds(cid * chunk, chunk)],
                             idx_v.at[b], isem[b])

        def drain_idx(b):
            pltpu.make_async_copy(idx_hbm.at[pl.ds(0, chunk)], idx_v.at[b],
                                  isem[b]).wait()

        def fire_in(cid, b):
            base = cid * chunk
            for o, sz in sub_offs:
                pltpu.async_copy(
                    poses_hbm.at[idx_v.at[b].at[pl.ds(o, sz)]],
                    poses_v.at[b].at[pl.ds(o, sz)],
                    gsem[b],
                )
            for c in range(3):
                pltpu.async_copy(ins[c].at[pl.ds(base, chunk)],
                                 pts_v.at[b].at[c], gsem[b])

        def drain_in(b):
            for o, sz in sub_offs:
                pltpu.make_async_copy(
                    poses_hbm.at[pl.ds(0, sz)],
                    poses_v.at[b].at[pl.ds(o, sz)],
                    gsem[b],
                ).wait()
            for c in range(3):
                pltpu.make_async_copy(ins[c].at[pl.ds(0, chunk)],
                                      pts_v.at[b].at[c], gsem[b]).wait()

        def fire_out(cid, b):
            pltpu.async_copy(out_v.at[b],
                             out_hbm.at[pl.ds(cid * tiles, tiles)],
                             osem[b])

        def drain_out(b):
            pltpu.make_async_copy(out_v.at[b],
                                  out_hbm.at[pl.ds(0, tiles)],
                                  osem[b]).wait()

        def compute(b):
            pv = poses_v.at[b]
            xr = pts_v.at[b].at[0]
            yr = pts_v.at[b].at[1]
            zr = pts_v.at[b].at[2]
            ov = out_v.at[b]

            def grp_body(g, _):
                rows = g * _L + lanes
                sp = pl.ds(g * _L, _L)
                t = g >> 3
                s = pl.ds((g & 7) * _L, _L)
                p = [plsc.load_gather(pv, [rows, col[j]]) for j in range(12)]
                x = xr[sp]
                y = yr[sp]
                z = zr[sp]
                ov[t, 0, s] = p[0] * x + p[1] * y + p[2] * z + p[9]
                ov[t, 1, s] = p[3] * x + p[4] * y + p[5] * z + p[10]
                ov[t, 2, s] = p[6] * x + p[7] * y + p[8] * z + p[11]
                return 0

            lax.fori_loop(0, n_grp, grp_body, 0, unroll=False)

        def process(cid, b):
            @pl.when(cid < n_chunks)
            def _():
                @pl.when(cid + 3 * _NW < n_chunks)
                def _():
                    fire_idx(cid + 3 * _NW, (b + 3) % 4)

                @pl.when(cid + 2 * _NW < n_chunks)
                def _():
                    drain_idx((b + 2) % 4)
                    fire_in(cid + 2 * _NW, (b + 2) % 4)

                drain_in(b)

                @pl.when(cid >= 4 * _NW)
                def _():
                    drain_out(b)

                compute(b)
                fire_out(cid, b)

        fire_idx(wid, 0)
        fire_idx(wid + _NW, 1)
        drain_idx(0)
        fire_in(wid, 0)
        drain_idx(1)
        fire_in(wid + _NW, 1)

        @pl.when(wid + 2 * _NW < n_chunks)
        def _():
            fire_idx(wid + 2 * _NW, 2)

        def quad_body(p, _):
            c_a = wid + (4 * p) * _NW
            process(c_a, 0)
            process(c_a + _NW, 1)
            process(c_a + 2 * _NW, 2)
            process(c_a + 3 * _NW, 3)
            return 0

        lax.fori_loop(0, n_quad, quad_body, 0, unroll=False)
        drain_out(0)
        drain_out(1)
        drain_out(2)
        drain_out(3)

    return apply


def kernel(frame_indices, pts_3d, rvecs, tvecs):
    n = rvecs.shape[0]
    m = pts_3d.shape[0]
    block = 4096
    n_pad = ((n + block - 1) // block) * block
    rv_t = jnp.pad(rvecs, ((0, n_pad - n), (0, 0))).T
    tv_t = jnp.pad(tvecs, ((0, n_pad - n), (0, 0))).T
    poses = _build_pose_table(rv_t, tv_t, n_pad, block)

    chunk = 640
    apply = _make_apply(m, chunk)
    out4 = apply(frame_indices.astype(jnp.int32),
                 pts_3d[:, 0], pts_3d[:, 1], pts_3d[:, 2], poses)
    out = out4.transpose(0, 2, 1)
    return out.reshape(m, 4)[:, :3]


# speedup vs baseline: 1.4542x; 1.1775x over previous
import functools

import jax
import jax.numpy as jnp
from jax import lax
from jax.experimental import pallas as pl
from jax.experimental.pallas import tpu as pltpu
from jax.experimental.pallas import tpu_sc as plsc

_NC = 2
_NS = 16
_NW = _NC * _NS
_L = 16

_POSE_W = 16


def _pose_table_body(rv_ref, tv_ref, out_ref):
    rx = rv_ref[0, :]
    ry = rv_ref[1, :]
    rz = rv_ref[2, :]
    theta = jnp.sqrt(rx * rx + ry * ry + rz * rz)
    safe = jnp.where(theta < 1e-8, jnp.float32(1.0), theta)
    inv = 1.0 / safe
    kx = rx * inv
    ky = ry * inv
    kz = rz * inv
    s = jnp.sin(safe)
    c = jnp.cos(safe)
    one_c = 1.0 - c
    kk = kx * kx + ky * ky + kz * kz
    r00 = 1.0 + one_c * (kx * kx - kk)
    r11 = 1.0 + one_c * (ky * ky - kk)
    r22 = 1.0 + one_c * (kz * kz - kk)
    xy = one_c * kx * ky
    xz = one_c * kx * kz
    yz = one_c * ky * kz
    skx = s * kx
    sky = s * ky
    skz = s * kz
    r01 = xy - skz
    r10 = xy + skz
    r02 = xz + sky
    r20 = xz - sky
    r12 = yz - skx
    r21 = yz + skx
    cols = [r00, r01, r02, r10, r11, r12, r20, r21, r22,
            tv_ref[0, :], tv_ref[1, :], tv_ref[2, :]]
    zero = jnp.zeros_like(rx)
    cols += [zero, zero, zero, zero]
    out_ref[...] = jnp.stack(cols, axis=0).T


def _build_pose_table(rv_t, tv_t, n_pad, block):
    grid = n_pad // block
    return pl.pallas_call(
        _pose_table_body,
        grid=(grid,),
        in_specs=[
            pl.BlockSpec((3, block), lambda i: (0, i)),
            pl.BlockSpec((3, block), lambda i: (0, i)),
        ],
        out_specs=pl.BlockSpec((block, _POSE_W), lambda i: (i, 0)),
        out_shape=jax.ShapeDtypeStruct((n_pad, _POSE_W), jnp.float32),
    )(rv_t, tv_t)


def _make_apply(m_points, chunk):
    assert m_points % chunk == 0 and chunk % 8 == 0
    n_chunks = m_points // chunk
    iters = (n_chunks + _NW - 1) // _NW
    n_quad = (iters + 3) // 4
    n_grp = chunk // _L
    sub_offs = [(o, min(128, chunk - o)) for o in range(0, chunk, 128)]
    mesh = plsc.VectorSubcoreMesh(core_axis_name="c", subcore_axis_name="s")

    assert chunk % 128 == 0

    tiles = chunk // 128

    @functools.partial(
        pl.kernel,
        out_type=jax.ShapeDtypeStruct((m_points // 128, 4, 128), jnp.float32),
        mesh=mesh,
        compiler_params=pltpu.CompilerParams(
            needs_layout_passes=False, use_tc_tiling_on_sc=False),
        scratch_types=[
            pltpu.VMEM((4, chunk), jnp.int32),
            pltpu.VMEM((4, chunk, _POSE_W), jnp.float32),
            pltpu.VMEM((4, tiles, 4, 128), jnp.float32),
            pltpu.VMEM((4, tiles, 4, 128), jnp.float32),
            [pltpu.SemaphoreType.DMA] * 4,
            [pltpu.SemaphoreType.DMA] * 4,
            [pltpu.SemaphoreType.DMA] * 4,
        ],
    )
    def apply(idx_hbm, pts_hbm, poses_hbm, out_hbm,
              idx_v, poses_v, pts_v, out_v, isem, gsem, osem):
        wid = lax.axis_index("s") * _NC + lax.axis_index("c")
        lanes = lax.iota(jnp.int32, _L)
        col = [jnp.full((_L,), j, jnp.int32) for j in range(12)]

        def fire_idx(cid, b):
            pltpu.async_copy(idx_hbm.at[pl.ds(cid * chunk, chunk)],
                             idx_v.at[b], isem[b])

        def drain_idx(b):
            pltpu.make_async_copy(idx_hbm.at[pl.ds(0, chunk)], idx_v.at[b],
                                  isem[b]).wait()

        def fire_in(cid, b):
            base = cid * chunk
            for o, sz in sub_offs:
                pltpu.async_copy(
                    poses_hbm.at[idx_v.at[b].at[pl.ds(o, sz)]],
                    poses_v.at[b].at[pl.ds(o, sz)],
                    gsem[b],
                )
            pltpu.async_copy(pts_hbm.at[pl.ds(cid * tiles, tiles)],
                             pts_v.at[b], gsem[b])

        def drain_in(b):
            for o, sz in sub_offs:
                pltpu.make_async_copy(
                    poses_hbm.at[pl.ds(0, sz)],
                    poses_v.at[b].at[pl.ds(o, sz)],
                    gsem[b],
                ).wait()
            pltpu.make_async_copy(pts_hbm.at[pl.ds(0, tiles)],
                                  pts_v.at[b], gsem[b]).wait()

        def fire_out(cid, b):
            pltpu.async_copy(out_v.at[b],
                             out_hbm.at[pl.ds(cid * tiles, tiles)],
                             osem[b])

        def drain_out(b):
            pltpu.make_async_copy(out_v.at[b],
                                  out_hbm.at[pl.ds(0, tiles)],
                                  osem[b]).wait()

        def compute(b):
            pv = poses_v.at[b]
            xv = pts_v.at[b]
            ov = out_v.at[b]

            def grp_body(g, _):
                rows = g * _L + lanes
                t = g >> 3
                s = pl.ds((g & 7) * _L, _L)
                p = [plsc.load_gather(pv, [rows, col[j]]) for j in range(12)]
                x = xv[t, 0, s]
                y = xv[t, 1, s]
                z = xv[t, 2, s]
                ov[t, 0, s] = p[0] * x + p[1] * y + p[2] * z + p[9]
                ov[t, 1, s] = p[3] * x + p[4] * y + p[5] * z + p[10]
                ov[t, 2, s] = p[6] * x + p[7] * y + p[8] * z + p[11]
                return 0

            lax.fori_loop(0, n_grp, grp_body, 0, unroll=False)

        def process(cid, b):
            @pl.when(cid < n_chunks)
            def _():
                @pl.when(cid + 3 * _NW < n_chunks)
                def _():
                    fire_idx(cid + 3 * _NW, (b + 3) % 4)

                @pl.when(cid + 2 * _NW < n_chunks)
                def _():
                    drain_idx((b + 2) % 4)
                    fire_in(cid + 2 * _NW, (b + 2) % 4)

                drain_in(b)

                @pl.when(cid >= 4 * _NW)
                def _():
                    drain_out(b)

                compute(b)
                fire_out(cid, b)

        fire_idx(wid, 0)
        fire_idx(wid + _NW, 1)
        drain_idx(0)
        fire_in(wid, 0)
        drain_idx(1)
        fire_in(wid + _NW, 1)

        @pl.when(wid + 2 * _NW < n_chunks)
        def _():
            fire_idx(wid + 2 * _NW, 2)

        def quad_body(p, _):
            c_a = wid + (4 * p) * _NW
            process(c_a, 0)
            process(c_a + _NW, 1)
            process(c_a + 2 * _NW, 2)
            process(c_a + 3 * _NW, 3)
            return 0

        lax.fori_loop(0, n_quad, quad_body, 0, unroll=False)
        drain_out(0)
        drain_out(1)
        drain_out(2)
        drain_out(3)

    return apply


def kernel(frame_indices, pts_3d, rvecs, tvecs):
    n = rvecs.shape[0]
    m = pts_3d.shape[0]
    block = 4096
    n_pad = ((n + block - 1) // block) * block
    rv_t = jnp.pad(rvecs, ((0, n_pad - n), (0, 0))).T
    tv_t = jnp.pad(tvecs, ((0, n_pad - n), (0, 0))).T
    poses = _build_pose_table(rv_t, tv_t, n_pad, block)

    pts4 = jnp.pad(pts_3d, ((0, 0), (0, 1)))
    pts_t = pts4.reshape(m // 128, 128, 4).transpose(0, 2, 1)

    chunk = 640
    apply = _make_apply(m, chunk)
    out4 = apply(frame_indices.astype(jnp.int32), pts_t, poses)
    out = out4.transpose(0, 2, 1)
    return out.reshape(m, 4)[:, :3]
